# baseline (speedup 1.0000x reference)
import functools

import jax
import jax.numpy as jnp
from jax import lax
from jax.experimental import pallas as pl
from jax.experimental.pallas import tpu as pltpu
from jax.experimental.pallas import tpu_sc as plsc

B = 16384
D = 208
NC = 2
NS = 16
NW = NC * NS
BPW = B // NW
CH = 128
NCH = BPW // CH
L = 16
NJ = D // L


def _tanh16(x):
    e = jnp.exp(x + x)
    return 1.0 - 2.0 / (e + 1.0)


def _precompute_body(e_ref, w1_ref, b1_ref, p1_ref, p2_ref):
    e = e_ref[...]
    p1 = jnp.dot(e, w1_ref[0:256, :], preferred_element_type=jnp.float32)
    p1 = p1 + b1_ref[...]
    p2 = jnp.dot(e, w1_ref[256:512, :], preferred_element_type=jnp.float32)
    pad = jnp.zeros((256, D - 200), jnp.float32)
    p1_ref[...] = jnp.concatenate([p1, pad], axis=1)
    p2_ref[...] = jnp.concatenate([p2, pad], axis=1)


def _precompute(embed_table, w1, b1_row):
    return pl.pallas_call(
        _precompute_body,
        out_shape=(
            jax.ShapeDtypeStruct((256, D), jnp.float32),
            jax.ShapeDtypeStruct((256, D), jnp.float32),
        ),
    )(embed_table, w1, b1_row)


def _sc_fused(p1_hbm, p2_hbm, inp_hbm, w2_hbm, b2_hbm, out_hbm,
              inp_v, idx0_v, idx1_v, r1a, r2a, r1b, r2b, w2_v, b2_v, out_v,
              sem1a, sem2a, sem1b, sem2b):
    wid = lax.axis_index("s") * NC + lax.axis_index("c")
    base = wid * BPW
    pltpu.sync_copy(inp_hbm.at[pl.ds(base, BPW)], inp_v)
    pltpu.sync_copy(w2_hbm, w2_v)
    pltpu.sync_copy(b2_hbm, b2_v)

    lane = lax.iota(jnp.int32, L)
    last_lane = lane == (L - 1)
    zeros16 = jnp.zeros((L,), jnp.int32)
    ones16 = zeros16 + 1

    for g in range(BPW // L):
        rows = jnp.full((L,), g * L, jnp.int32) + lane
        idx0_v[pl.ds(g * L, L)] = plsc.load_gather(inp_v, [rows, zeros16])
        idx1_v[pl.ds(g * L, L)] = plsc.load_gather(inp_v, [rows, ones16])

    w2r = [w2_v[pl.ds(j * L, L)] for j in range(NJ)]

    bufs = [(r1a, r2a, sem1a, sem2a), (r1b, r2b, sem1b, sem2b)]

    def start(c):
        r1, r2, s1, s2 = bufs[c % 2]
        cp1 = pltpu.async_copy(p1_hbm.at[idx0_v.at[pl.ds(c * CH, CH)]], r1, s1)
        cp2 = pltpu.async_copy(p2_hbm.at[idx1_v.at[pl.ds(c * CH, CH)]], r2, s2)
        return cp1, cp2

    pending = {0: start(0)}
    for c in range(NCH):
        if c + 1 < NCH:
            pending[c + 1] = start(c + 1)
        cp1, cp2 = pending.pop(c)
        cp1.wait()
        cp2.wait()
        r1, r2, _, _ = bufs[c % 2]

        def row_body(i, carry):
            acc = jnp.zeros((L,), jnp.float32)
            for j in range(NJ):
                sl = pl.ds(j * L, L)
                a1 = _tanh16(r1[i, sl] + r2[i, sl])
                acc = acc + a1 * w2r[j]
            total = plsc.cumsum(acc)
            plsc.store_scatter(out_v, [jnp.full((L,), c * CH, jnp.int32) + i],
                               total, mask=last_lane)
            return carry

        lax.fori_loop(0, CH, row_body, 0)

    b2r = b2_v[...]
    for g in range(BPW // L):
        sl = pl.ds(g * L, L)
        out_v[sl] = _tanh16(out_v[sl] + b2r)
    pltpu.sync_copy(out_v, out_hbm.at[pl.ds(base, BPW)])


_sc_fused_call = functools.partial(
    pl.kernel,
    out_type=jax.ShapeDtypeStruct((B,), jnp.float32),
    mesh=plsc.VectorSubcoreMesh(core_axis_name="c", subcore_axis_name="s"),
    compiler_params=pltpu.CompilerParams(use_tc_tiling_on_sc=False,
                                         needs_layout_passes=False),
    scratch_types=[
        pltpu.VMEM((BPW, 2), jnp.int32),
        pltpu.VMEM((BPW,), jnp.int32),
        pltpu.VMEM((BPW,), jnp.int32),
        pltpu.VMEM((CH, D), jnp.float32),
        pltpu.VMEM((CH, D), jnp.float32),
        pltpu.VMEM((CH, D), jnp.float32),
        pltpu.VMEM((CH, D), jnp.float32),
        pltpu.VMEM((D,), jnp.float32),
        pltpu.VMEM((L,), jnp.float32),
        pltpu.VMEM((BPW,), jnp.float32),
        pltpu.SemaphoreType.DMA,
        pltpu.SemaphoreType.DMA,
        pltpu.SemaphoreType.DMA,
        pltpu.SemaphoreType.DMA,
    ],
)(_sc_fused)


def kernel(inp, embed_table, W1, b1, W2, b2):
    p1, p2 = _precompute(embed_table, W1, b1.reshape(1, 200))
    w2_pad = jnp.pad(W2[:, 0], (0, D - 200))
    b2_vec = jnp.broadcast_to(b2.astype(jnp.float32), (L,))
    out = _sc_fused_call(p1, p2, inp.astype(jnp.int32), w2_pad, b2_vec)
    return out.reshape(B, 1)


# speedup vs baseline: 1.2836x; 1.2836x over previous
import functools

import jax
import jax.numpy as jnp
from jax import lax
from jax.experimental import pallas as pl
from jax.experimental.pallas import tpu as pltpu
from jax.experimental.pallas import tpu_sc as plsc

B = 16384
D = 208
NC = 2
NS = 16
NW = NC * NS
BPW = B // NW
CH = 128
NCH = BPW // CH
L = 16
NJ = D // L


def _tanh16(x):
    e = jnp.exp(x + x)
    return 1.0 - 2.0 / (e + 1.0)


def _precompute_body(e_ref, w1_ref, b1_ref, p1_ref, p2_ref):
    e = e_ref[...]
    p1 = jnp.dot(e, w1_ref[0:256, :], preferred_element_type=jnp.float32)
    p1 = p1 + b1_ref[...]
    p2 = jnp.dot(e, w1_ref[256:512, :], preferred_element_type=jnp.float32)
    pad = jnp.zeros((256, D - 200), jnp.float32)
    p1_ref[...] = jnp.concatenate([p1, pad], axis=1)
    p2_ref[...] = jnp.concatenate([p2, pad], axis=1)


def _precompute(embed_table, w1, b1_row):
    return pl.pallas_call(
        _precompute_body,
        out_shape=(
            jax.ShapeDtypeStruct((256, D), jnp.float32),
            jax.ShapeDtypeStruct((256, D), jnp.float32),
        ),
    )(embed_table, w1, b1_row)


def _sc_fused(p1_hbm, p2_hbm, idx0_hbm, idx1_hbm, w2_hbm, b2_hbm, out_hbm,
              idx0_v, idx1_v, r1a, r2a, r1b, r2b, w2_v, b2_v, out_v,
              sem1a, sem2a, sem1b, sem2b):
    wid = lax.axis_index("s") * NC + lax.axis_index("c")
    base = wid * BPW
    pltpu.sync_copy(idx0_hbm.at[pl.ds(base, BPW)], idx0_v)
    pltpu.sync_copy(idx1_hbm.at[pl.ds(base, BPW)], idx1_v)
    pltpu.sync_copy(w2_hbm, w2_v)
    pltpu.sync_copy(b2_hbm, b2_v)

    lane = lax.iota(jnp.int32, L)
    last_lane = lane == (L - 1)

    w2r = [w2_v[pl.ds(j * L, L)] for j in range(NJ)]

    bufs = [(r1a, r2a, sem1a, sem2a), (r1b, r2b, sem1b, sem2b)]

    def start(c):
        r1, r2, s1, s2 = bufs[c % 2]
        cp1 = pltpu.async_copy(p1_hbm.at[idx0_v.at[pl.ds(c * CH, CH)]], r1, s1)
        cp2 = pltpu.async_copy(p2_hbm.at[idx1_v.at[pl.ds(c * CH, CH)]], r2, s2)
        return cp1, cp2

    pending = {0: start(0)}
    for c in range(NCH):
        if c + 1 < NCH:
            pending[c + 1] = start(c + 1)
        cp1, cp2 = pending.pop(c)
        cp1.wait()
        cp2.wait()
        r1, r2, _, _ = bufs[c % 2]

        def row_body(i, carry):
            acc = jnp.zeros((L,), jnp.float32)
            for j in range(NJ):
                sl = pl.ds(j * L, L)
                a1 = _tanh16(r1[i, sl] + r2[i, sl])
                acc = acc + a1 * w2r[j]
            total = plsc.cumsum(acc)
            plsc.store_scatter(out_v, [jnp.full((L,), c * CH, jnp.int32) + i],
                               total, mask=last_lane)
            return carry

        lax.fori_loop(0, CH, row_body, 0)

    b2r = b2_v[...]
    for g in range(BPW // L):
        sl = pl.ds(g * L, L)
        out_v[sl] = _tanh16(out_v[sl] + b2r)
    pltpu.sync_copy(out_v, out_hbm.at[pl.ds(base, BPW)])


_sc_fused_call = functools.partial(
    pl.kernel,
    out_type=jax.ShapeDtypeStruct((B,), jnp.float32),
    mesh=plsc.VectorSubcoreMesh(core_axis_name="c", subcore_axis_name="s"),
    compiler_params=pltpu.CompilerParams(use_tc_tiling_on_sc=False,
                                         needs_layout_passes=False),
    scratch_types=[
        pltpu.VMEM((BPW,), jnp.int32),
        pltpu.VMEM((BPW,), jnp.int32),
        pltpu.VMEM((CH, D), jnp.float32),
        pltpu.VMEM((CH, D), jnp.float32),
        pltpu.VMEM((CH, D), jnp.float32),
        pltpu.VMEM((CH, D), jnp.float32),
        pltpu.VMEM((D,), jnp.float32),
        pltpu.VMEM((L,), jnp.float32),
        pltpu.VMEM((BPW,), jnp.float32),
        pltpu.SemaphoreType.DMA,
        pltpu.SemaphoreType.DMA,
        pltpu.SemaphoreType.DMA,
        pltpu.SemaphoreType.DMA,
    ],
)(_sc_fused)


def kernel(inp, embed_table, W1, b1, W2, b2):
    idx = inp.astype(jnp.int32)
    p1, p2 = _precompute(embed_table, W1, b1.reshape(1, 200))
    w2_pad = jnp.pad(W2[:, 0], (0, D - 200))
    b2_vec = jnp.broadcast_to(b2.astype(jnp.float32), (L,))
    out = _sc_fused_call(p1, p2, idx[:, 0], idx[:, 1], w2_pad, b2_vec)
    return out.reshape(B, 1)


# speedup vs baseline: 1.5369x; 1.1974x over previous
import functools

import jax
import jax.numpy as jnp
from jax import lax
from jax.experimental import pallas as pl
from jax.experimental.pallas import tpu as pltpu
from jax.experimental.pallas import tpu_sc as plsc

B = 16384
D = 208
NC = 2
NS = 16
NW = NC * NS
BPW = B // NW
CH = 128
NCH = BPW // CH
L = 16
NJ = D // L


def _tanh16(x):
    e = jnp.exp(x + x)
    return 1.0 - 2.0 / (e + 1.0)


_TWO_LOG2E = 2.0


def _precompute_body(e_ref, w1_ref, b1_ref, p1_ref, p2_ref):
    e = e_ref[...]
    p1 = jnp.dot(e, w1_ref[0:256, :], preferred_element_type=jnp.float32)
    p1 = (p1 + b1_ref[...]) * _TWO_LOG2E
    p2 = jnp.dot(e, w1_ref[256:512, :], preferred_element_type=jnp.float32)
    p2 = p2 * _TWO_LOG2E
    pad = jnp.zeros((256, D - 200), jnp.float32)
    p1_ref[...] = jnp.concatenate([p1, pad], axis=1)
    p2_ref[...] = jnp.concatenate([p2, pad], axis=1)


def _precompute(embed_table, w1, b1_row):
    return pl.pallas_call(
        _precompute_body,
        out_shape=(
            jax.ShapeDtypeStruct((256, D), jnp.float32),
            jax.ShapeDtypeStruct((256, D), jnp.float32),
        ),
    )(embed_table, w1, b1_row)


def _sc_fused(p1_hbm, p2_hbm, idx0_hbm, idx1_hbm, w2_hbm, b2_hbm, out_hbm,
              idx0_v, idx1_v, r1a, r2a, r1b, r2b, w2_v, b2_v, out_v,
              sem1a, sem2a, sem1b, sem2b):
    wid = lax.axis_index("s") * NC + lax.axis_index("c")
    base = wid * BPW
    pltpu.sync_copy(idx0_hbm.at[pl.ds(base, BPW)], idx0_v)
    pltpu.sync_copy(idx1_hbm.at[pl.ds(base, BPW)], idx1_v)
    pltpu.sync_copy(w2_hbm, w2_v)
    pltpu.sync_copy(b2_hbm, b2_v)

    lane = lax.iota(jnp.int32, L)
    last_lane = lane == (L - 1)

    w2r = [w2_v[pl.ds(j * L, L)] for j in range(NJ)]

    bufs = [(r1a, r2a, sem1a, sem2a), (r1b, r2b, sem1b, sem2b)]

    def start(c):
        r1, r2, s1, s2 = bufs[c % 2]
        cp1 = pltpu.async_copy(p1_hbm.at[idx0_v.at[pl.ds(c * CH, CH)]], r1, s1)
        cp2 = pltpu.async_copy(p2_hbm.at[idx1_v.at[pl.ds(c * CH, CH)]], r2, s2)
        return cp1, cp2

    pending = {0: start(0)}
    for c in range(NCH):
        if c + 1 < NCH:
            pending[c + 1] = start(c + 1)
        cp1, cp2 = pending.pop(c)
        cp1.wait()
        cp2.wait()
        r1, r2, _, _ = bufs[c % 2]

        def row_body(i):
            acc = jnp.zeros((L,), jnp.float32)
            for j in range(NJ):
                sl = pl.ds(j * L, L)
                e = jnp.exp(r1[i, sl] + r2[i, sl])
                acc = acc + w2r[j] / (e + 1.0)
            total = plsc.cumsum(acc)
            plsc.store_scatter(out_v, [jnp.full((L,), c * CH, jnp.int32) + i],
                               total, mask=last_lane)

        plsc.parallel_loop(0, CH, 1, unroll=2)(row_body)

    b2r = b2_v[...]
    for g in range(BPW // L):
        sl = pl.ds(g * L, L)
        out_v[sl] = _tanh16(b2r - out_v[sl])
    pltpu.sync_copy(out_v, out_hbm.at[pl.ds(base, BPW)])


_sc_fused_call = functools.partial(
    pl.kernel,
    out_type=jax.ShapeDtypeStruct((B,), jnp.float32),
    mesh=plsc.VectorSubcoreMesh(core_axis_name="c", subcore_axis_name="s"),
    compiler_params=pltpu.CompilerParams(use_tc_tiling_on_sc=False,
                                         needs_layout_passes=False),
    scratch_types=[
        pltpu.VMEM((BPW,), jnp.int32),
        pltpu.VMEM((BPW,), jnp.int32),
        pltpu.VMEM((CH, D), jnp.float32),
        pltpu.VMEM((CH, D), jnp.float32),
        pltpu.VMEM((CH, D), jnp.float32),
        pltpu.VMEM((CH, D), jnp.float32),
        pltpu.VMEM((D,), jnp.float32),
        pltpu.VMEM((L,), jnp.float32),
        pltpu.VMEM((BPW,), jnp.float32),
        pltpu.SemaphoreType.DMA,
        pltpu.SemaphoreType.DMA,
        pltpu.SemaphoreType.DMA,
        pltpu.SemaphoreType.DMA,
    ],
)(_sc_fused)


def kernel(inp, embed_table, W1, b1, W2, b2):
    idx = inp.astype(jnp.int32)
    p1, p2 = _precompute(embed_table, W1, b1.reshape(1, 200))
    w2_pad = jnp.pad(2.0 * W2[:, 0], (0, D - 200))
    b2_vec = jnp.broadcast_to(b2.astype(jnp.float32) + jnp.sum(W2), (L,))
    out = _sc_fused_call(p1, p2, idx[:, 0], idx[:, 1], w2_pad, b2_vec)
    return out.reshape(B, 1)


# speedup vs baseline: 1.5527x; 1.0103x over previous
import functools

import jax
import jax.numpy as jnp
from jax import lax
from jax.experimental import pallas as pl
from jax.experimental.pallas import tpu as pltpu
from jax.experimental.pallas import tpu_sc as plsc

B = 16384
D = 208
NC = 2
NS = 16
NW = NC * NS
BPW = B // NW
CH = 128
NCH = BPW // CH
L = 16
NJ = D // L


def _tanh16(x):
    e = jnp.exp(x + x)
    return 1.0 - 2.0 / (e + 1.0)


_TWO_LOG2E = 2.0


def _precompute_body(e_ref, w1_ref, b1_ref, p1_ref, p2_ref):
    e = e_ref[...]
    p1 = jnp.dot(e, w1_ref[0:256, :], preferred_element_type=jnp.float32)
    p1 = (p1 + b1_ref[...]) * _TWO_LOG2E
    p2 = jnp.dot(e, w1_ref[256:512, :], preferred_element_type=jnp.float32)
    p2 = p2 * _TWO_LOG2E
    pad = jnp.zeros((256, D - 200), jnp.float32)
    p1_ref[...] = jnp.concatenate([p1, pad], axis=1)
    p2_ref[...] = jnp.concatenate([p2, pad], axis=1)


def _precompute(embed_table, w1, b1_row):
    return pl.pallas_call(
        _precompute_body,
        out_shape=(
            jax.ShapeDtypeStruct((256, D), jnp.float32),
            jax.ShapeDtypeStruct((256, D), jnp.float32),
        ),
    )(embed_table, w1, b1_row)


def _sc_fused(p1_hbm, p2_hbm, idx0_hbm, idx1_hbm, w2_hbm, b2_hbm, out_hbm,
              idx0_v, idx1_v, r1a, r2a, r1b, r2b, w2_v, b2_v, out_v,
              sem1a, sem2a, sem1b, sem2b):
    wid = lax.axis_index("s") * NC + lax.axis_index("c")
    base = wid * BPW
    pltpu.sync_copy(idx0_hbm.at[pl.ds(base, BPW)], idx0_v)
    pltpu.sync_copy(idx1_hbm.at[pl.ds(base, BPW)], idx1_v)
    pltpu.sync_copy(w2_hbm, w2_v)
    pltpu.sync_copy(b2_hbm, b2_v)

    lane = lax.iota(jnp.int32, L)
    last_lane = lane == (L - 1)

    w2r = [w2_v[pl.ds(j * L, L)] for j in range(NJ)]

    bufs = [(r1a, r2a, sem1a, sem2a), (r1b, r2b, sem1b, sem2b)]

    def start(c, bi):
        r1, r2, s1, s2 = bufs[bi]
        pltpu.async_copy(p1_hbm.at[idx0_v.at[pl.ds(c * CH, CH)]], r1, s1)
        pltpu.async_copy(p2_hbm.at[idx1_v.at[pl.ds(c * CH, CH)]], r2, s2)

    def wait(bi):
        r1, r2, s1, s2 = bufs[bi]
        pltpu.make_async_copy(p1_hbm.at[idx0_v.at[pl.ds(0, CH)]], r1, s1).wait()
        pltpu.make_async_copy(p2_hbm.at[idx1_v.at[pl.ds(0, CH)]], r2, s2).wait()

    def compute(c, bi):
        r1, r2, _, _ = bufs[bi]

        def row_body(i):
            acc = jnp.zeros((L,), jnp.float32)
            for j in range(NJ):
                sl = pl.ds(j * L, L)
                e = jnp.exp(r1[i, sl] + r2[i, sl])
                acc = acc + w2r[j] / (e + 1.0)
            total = plsc.cumsum(acc)
            plsc.store_scatter(out_v, [jnp.full((L,), c * CH, jnp.int32) + i],
                               total, mask=last_lane)

        plsc.parallel_loop(0, CH, 1, unroll=2)(row_body)

    start(0, 0)

    def pair_body(t, carry):
        c0 = 2 * t
        start(c0 + 1, 1)
        wait(0)
        compute(c0, 0)

        @pl.when(t + 1 < NCH // 2)
        def _():
            start(c0 + 2, 0)

        wait(1)
        compute(c0 + 1, 1)
        return carry

    lax.fori_loop(0, NCH // 2, pair_body, 0)

    b2r = b2_v[...]
    for g in range(BPW // L):
        sl = pl.ds(g * L, L)
        out_v[sl] = _tanh16(b2r - out_v[sl])
    pltpu.sync_copy(out_v, out_hbm.at[pl.ds(base, BPW)])


_sc_fused_call = functools.partial(
    pl.kernel,
    out_type=jax.ShapeDtypeStruct((B,), jnp.float32),
    mesh=plsc.VectorSubcoreMesh(core_axis_name="c", subcore_axis_name="s"),
    compiler_params=pltpu.CompilerParams(use_tc_tiling_on_sc=False,
                                         needs_layout_passes=False),
    scratch_types=[
        pltpu.VMEM((BPW,), jnp.int32),
        pltpu.VMEM((BPW,), jnp.int32),
        pltpu.VMEM((CH, D), jnp.float32),
        pltpu.VMEM((CH, D), jnp.float32),
        pltpu.VMEM((CH, D), jnp.float32),
        pltpu.VMEM((CH, D), jnp.float32),
        pltpu.VMEM((D,), jnp.float32),
        pltpu.VMEM((L,), jnp.float32),
        pltpu.VMEM((BPW,), jnp.float32),
        pltpu.SemaphoreType.DMA,
        pltpu.SemaphoreType.DMA,
        pltpu.SemaphoreType.DMA,
        pltpu.SemaphoreType.DMA,
    ],
)(_sc_fused)


def kernel(inp, embed_table, W1, b1, W2, b2):
    idx = inp.astype(jnp.int32)
    p1, p2 = _precompute(embed_table, W1, b1.reshape(1, 200))
    w2_pad = jnp.pad(2.0 * W2[:, 0], (0, D - 200))
    b2_vec = jnp.broadcast_to(b2.astype(jnp.float32) + jnp.sum(W2), (L,))
    out = _sc_fused_call(p1, p2, idx[:, 0], idx[:, 1], w2_pad, b2_vec)
    return out.reshape(B, 1)


# speedup vs baseline: 1.6176x; 1.0418x over previous
import functools

import jax
import jax.numpy as jnp
from jax import lax
from jax.experimental import pallas as pl
from jax.experimental.pallas import tpu as pltpu
from jax.experimental.pallas import tpu_sc as plsc

B = 16384
D = 208
TR = 514
NC = 2
NS = 16
NW = NC * NS
BPW = B // NW
CH = 128
NCH = BPW // CH
L = 16
NJ = D // L


def _tanh16(x):
    e = jnp.exp(x + x)
    return 1.0 - 2.0 / (e + 1.0)


def _precompute_body(e_ref, w1_ref, b1_ref, w2_ref, b2_ref, t_ref):
    e = e_ref[...]
    p1 = jnp.dot(e, w1_ref[0:256, :], preferred_element_type=jnp.float32)
    p1 = (p1 + b1_ref[...]) * 2.0
    p2 = jnp.dot(e, w1_ref[256:512, :], preferred_element_type=jnp.float32)
    p2 = p2 * 2.0
    pad = jnp.zeros((256, D - 200), jnp.float32)
    t_ref[0:256, :] = jnp.concatenate([p1, pad], axis=1)
    t_ref[256:512, :] = jnp.concatenate([p2, pad], axis=1)
    w2 = w2_ref[...]
    w2row = jnp.concatenate([2.0 * w2, jnp.zeros((1, D - 200), jnp.float32)],
                            axis=1)
    t_ref[512:513, :] = w2row
    bias = b2_ref[0] + jnp.sum(w2)
    t_ref[513:514, :] = jnp.full((1, D), bias, jnp.float32)


def _precompute(embed_table, w1, b1_row, w2_row, b2):
    return pl.pallas_call(
        _precompute_body,
        in_specs=[
            pl.BlockSpec((256, 256), lambda: (0, 0)),
            pl.BlockSpec((512, 200), lambda: (0, 0)),
            pl.BlockSpec((1, 200), lambda: (0, 0)),
            pl.BlockSpec((1, 200), lambda: (0, 0)),
            pl.BlockSpec(memory_space=pltpu.SMEM),
        ],
        out_specs=pl.BlockSpec((TR, D), lambda: (0, 0)),
        out_shape=jax.ShapeDtypeStruct((TR, D), jnp.float32),
    )(embed_table, w1, b1_row, w2_row, b2)


def _sc_fused(t_hbm, idx0_hbm, idx1_hbm, out_hbm,
              idx0_v, idx1_v, r1a, r2a, r1b, r2b, wb_v, out_v,
              sem1a, sem2a, sem1b, sem2b):
    wid = lax.axis_index("s") * NC + lax.axis_index("c")
    base = wid * BPW
    pltpu.sync_copy(idx0_hbm.at[pl.ds(base, BPW)], idx0_v)
    pltpu.sync_copy(idx1_hbm.at[pl.ds(base, BPW)], idx1_v)
    pltpu.sync_copy(t_hbm.at[pl.ds(512, 2)], wb_v)

    lane = lax.iota(jnp.int32, L)
    last_lane = lane == (L - 1)

    w2r = [wb_v[0, pl.ds(j * L, L)] for j in range(NJ)]
    b2r = wb_v[1, pl.ds(0, L)]

    bufs = [(r1a, r2a, sem1a, sem2a), (r1b, r2b, sem1b, sem2b)]

    def start(c, bi):
        r1, r2, s1, s2 = bufs[bi]
        pltpu.async_copy(t_hbm.at[idx0_v.at[pl.ds(c * CH, CH)]], r1, s1)
        pltpu.async_copy(t_hbm.at[idx1_v.at[pl.ds(c * CH, CH)]], r2, s2)

    def wait(bi):
        r1, r2, s1, s2 = bufs[bi]
        pltpu.make_async_copy(t_hbm.at[idx0_v.at[pl.ds(0, CH)]], r1, s1).wait()
        pltpu.make_async_copy(t_hbm.at[idx1_v.at[pl.ds(0, CH)]], r2, s2).wait()

    def compute(c, bi):
        r1, r2, _, _ = bufs[bi]

        def row_body(i):
            acc = jnp.zeros((L,), jnp.float32)
            for j in range(NJ):
                sl = pl.ds(j * L, L)
                e = jnp.exp(r1[i, sl] + r2[i, sl])
                acc = acc + w2r[j] / (e + 1.0)
            total = plsc.cumsum(acc)
            plsc.store_scatter(out_v, [jnp.full((L,), c * CH, jnp.int32) + i],
                               total, mask=last_lane)

        plsc.parallel_loop(0, CH, 1, unroll=4)(row_body)

    start(0, 0)

    def pair_body(t, carry):
        c0 = 2 * t
        start(c0 + 1, 1)
        wait(0)
        compute(c0, 0)

        @pl.when(t + 1 < NCH // 2)
        def _():
            start(c0 + 2, 0)

        wait(1)
        compute(c0 + 1, 1)
        return carry

    lax.fori_loop(0, NCH // 2, pair_body, 0)

    for g in range(BPW // L):
        sl = pl.ds(g * L, L)
        out_v[sl] = _tanh16(b2r - out_v[sl])
    pltpu.sync_copy(out_v, out_hbm.at[pl.ds(base, BPW)])


_sc_fused_call = functools.partial(
    pl.kernel,
    out_type=jax.ShapeDtypeStruct((B,), jnp.float32),
    mesh=plsc.VectorSubcoreMesh(core_axis_name="c", subcore_axis_name="s"),
    compiler_params=pltpu.CompilerParams(use_tc_tiling_on_sc=False,
                                         needs_layout_passes=False),
    scratch_types=[
        pltpu.VMEM((BPW,), jnp.int32),
        pltpu.VMEM((BPW,), jnp.int32),
        pltpu.VMEM((CH, D), jnp.float32),
        pltpu.VMEM((CH, D), jnp.float32),
        pltpu.VMEM((CH, D), jnp.float32),
        pltpu.VMEM((CH, D), jnp.float32),
        pltpu.VMEM((2, D), jnp.float32),
        pltpu.VMEM((BPW,), jnp.float32),
        pltpu.SemaphoreType.DMA,
        pltpu.SemaphoreType.DMA,
        pltpu.SemaphoreType.DMA,
        pltpu.SemaphoreType.DMA,
    ],
)(_sc_fused)


def kernel(inp, embed_table, W1, b1, W2, b2):
    idx = inp.astype(jnp.int32)
    idx0 = idx[:, 0]
    idx1 = idx[:, 1] + 256
    table = _precompute(embed_table, W1, b1.reshape(1, 200),
                        W2.reshape(1, 200), b2.reshape(1))
    out = _sc_fused_call(table, idx0, idx1)
    return out.reshape(B, 1)
